# baseline (speedup 1.0000x reference)
import jax
import jax.numpy as jnp
from jax.experimental import pallas as pl

N = 4000
F = 256
HID = 128
NCLS = 16
KSEL = 50
HALF = F // 2
R = 400
G = N // R
BISECT_ITERS = 48


def _deg_body(adj_ref, dinv_ref):
    d = jnp.sum(adj_ref[...], axis=1, keepdims=True)
    dinv_ref[...] = jnp.where(d > 0, 1.0 / jnp.sqrt(d), 0.0)


def _enc1_body(adj_ref, x_ref, t1_ref, dinv_ref, dinvT_ref, w1_ref, h_ref, z1_ref):
    i = pl.program_id(0)
    x_blk = x_ref[pl.ds(i * R, R), :]
    z1_ref[...] = jnp.dot(x_blk, w1_ref[...], preferred_element_type=jnp.float32)
    normA = adj_ref[...] * dinv_ref[pl.ds(i * R, R), :] * dinvT_ref[...]
    y = x_ref[...] * t1_ref[...]
    h_ref[...] = jnp.tanh(jnp.dot(normA, y, preferred_element_type=jnp.float32))


def _enc2_body(adj_ref, h_ref, t2_ref, dinv_ref, dinvT_ref, emb_ref):
    i = pl.program_id(0)
    normA = adj_ref[...] * dinv_ref[pl.ds(i * R, R), :] * dinvT_ref[...]
    y = h_ref[...] * t2_ref[...]
    z = jnp.dot(normA, y, preferred_element_type=jnp.float32)
    nrm = jnp.sqrt(jnp.sum(z * z, axis=1, keepdims=True)) + 1e-12
    emb_ref[...] = z / nrm


def _sim_block(e, embT_ref):
    return (jnp.dot(e[:, :HALF], embT_ref[:HALF, :], preferred_element_type=jnp.float32)
            + jnp.dot(e[:, HALF:], embT_ref[HALF:, :], preferred_element_type=jnp.float32))


def _thr_body(emb_ref, embT_ref, t_ref):
    s = _sim_block(emb_ref[...], embT_ref)
    ones = jnp.ones((N, 1), jnp.float32)
    lo = jnp.full((R, 1), -2.0, jnp.float32)
    hi = jnp.full((R, 1), 2.0, jnp.float32)
    def body(_, carry):
        lo, hi = carry
        w = 0.25 * (hi - lo)
        m1 = lo + w
        m2 = lo + 2.0 * w
        m3 = lo + 3.0 * w
        c1 = jnp.dot(jnp.where(s >= m1, 1.0, 0.0), ones,
                     preferred_element_type=jnp.float32)
        c2 = jnp.dot(jnp.where(s >= m2, 1.0, 0.0), ones,
                     preferred_element_type=jnp.float32)
        c3 = jnp.dot(jnp.where(s >= m3, 1.0, 0.0), ones,
                     preferred_element_type=jnp.float32)
        g1 = c1 >= KSEL - 0.5
        g2 = c2 >= KSEL - 0.5
        g3 = c3 >= KSEL - 0.5
        lo2 = jnp.where(g3, m3, jnp.where(g2, m2, jnp.where(g1, m1, lo)))
        hi2 = jnp.where(g3, hi, jnp.where(g2, m3, jnp.where(g1, m2, m1)))
        return lo2, hi2

    lo, hi = jax.lax.fori_loop(0, 20, body, (lo, hi))
    t_ref[...] = lo


def _assemble_body(emb_ref, embT_ref, t_ref, tT_ref, conf_ref, th_ref, adj_ref,
                   anew_ref, afin_ref, afinb_ref, dinvf_ref):
    s = _sim_block(emb_ref[...], embT_ref)
    m = 0.5 * ((s >= t_ref[...]).astype(jnp.float32)
               + (s >= tT_ref[...]).astype(jnp.float32))
    a1 = s * m
    z = conf_ref[...] * (a1 > 0).astype(jnp.float32) - th_ref[...]
    w = 2.0 / (1.0 + jnp.exp(-z))
    mask = jnp.where(z >= 0, w, 0.5)
    anew = a1 * mask
    anew_ref[...] = anew
    afin = anew + adj_ref[...]
    afin_ref[...] = afin
    afinb_ref[...] = afin.astype(jnp.bfloat16)
    df = jnp.sum(afin, axis=1, keepdims=True)
    dinvf_ref[...] = jnp.where(df > 0, 1.0 / jnp.sqrt(df), 0.0)


def _task1_body(adjfb_ref, z1_ref, dinvf_ref, b1_ref, w2_ref, v_ref):
    i = pl.program_id(0)
    y = (z1_ref[...] * dinvf_ref[...]).astype(jnp.bfloat16)
    x1 = (jnp.dot(adjfb_ref[...], y, preferred_element_type=jnp.float32)
          * dinvf_ref[pl.ds(i * R, R), :] + b1_ref[...])
    x1 = jnp.maximum(x1, 0.0)
    v_ref[...] = jnp.dot(x1, w2_ref[...], preferred_element_type=jnp.float32)


def _task2_body(adjfb_ref, v_ref, dinvf_ref, b2_ref, out_ref):
    i = pl.program_id(0)
    y = (v_ref[...] * dinvf_ref[...]).astype(jnp.bfloat16)
    out_ref[...] = (jnp.dot(adjfb_ref[...], y, preferred_element_type=jnp.float32)
                    * dinvf_ref[pl.ds(i * R, R), :] + b2_ref[...])


def _blk(shape, imap):
    return pl.BlockSpec(shape, imap)


_ROW = lambda i: (i, 0)
_FULL = lambda i: (0, 0)


def kernel(input, Adj, t1, t2, thresholds, confidence_vector, W1, b1, W2, b2):
    f32 = jnp.float32
    t1r = t1.reshape(1, F)
    t2r = t2.reshape(1, F)
    b1r = b1.reshape(1, HID)
    b2r = b2.reshape(1, NCLS)
    confr = confidence_vector.reshape(1, N)

    dinv = pl.pallas_call(
        _deg_body, grid=(G,),
        in_specs=[_blk((R, N), _ROW)],
        out_specs=_blk((R, 1), _ROW),
        out_shape=jax.ShapeDtypeStruct((N, 1), f32),
    )(Adj)
    dinvT = dinv.reshape(1, N)

    h, z1 = pl.pallas_call(
        _enc1_body, grid=(G,),
        in_specs=[_blk((R, N), _ROW), _blk((N, F), _FULL), _blk((1, F), _FULL),
                  _blk((N, 1), _FULL), _blk((1, N), _FULL), _blk((F, HID), _FULL)],
        out_specs=[_blk((R, F), _ROW), _blk((R, HID), _ROW)],
        out_shape=[jax.ShapeDtypeStruct((N, F), f32),
                   jax.ShapeDtypeStruct((N, HID), f32)],
    )(Adj, input, t1r, dinv, dinvT, W1)

    emb = pl.pallas_call(
        _enc2_body, grid=(G,),
        in_specs=[_blk((R, N), _ROW), _blk((N, F), _FULL), _blk((1, F), _FULL),
                  _blk((N, 1), _FULL), _blk((1, N), _FULL)],
        out_specs=_blk((R, F), _ROW),
        out_shape=jax.ShapeDtypeStruct((N, F), f32),
    )(Adj, h, t2r, dinv, dinvT)
    embT = emb.T

    t = pl.pallas_call(
        _thr_body, grid=(G,),
        in_specs=[_blk((R, F), _ROW), _blk((F, N), _FULL)],
        out_specs=_blk((R, 1), _ROW),
        out_shape=jax.ShapeDtypeStruct((N, 1), f32),
    )(emb, embT)
    tT = t.reshape(1, N)

    anew, afin, afinb, dinvf = pl.pallas_call(
        _assemble_body, grid=(G,),
        in_specs=[_blk((R, F), _ROW), _blk((F, N), _FULL), _blk((R, 1), _ROW),
                  _blk((1, N), _FULL), _blk((1, N), _FULL), _blk((R, 1), _ROW),
                  _blk((R, N), _ROW)],
        out_specs=[_blk((R, N), _ROW), _blk((R, N), _ROW), _blk((R, N), _ROW),
                   _blk((R, 1), _ROW)],
        out_shape=[jax.ShapeDtypeStruct((N, N), f32),
                   jax.ShapeDtypeStruct((N, N), f32),
                   jax.ShapeDtypeStruct((N, N), jnp.bfloat16),
                   jax.ShapeDtypeStruct((N, 1), f32)],
    )(emb, embT, t, tT, confr, thresholds, Adj)

    v = pl.pallas_call(
        _task1_body, grid=(G,),
        in_specs=[_blk((R, N), _ROW), _blk((N, HID), _FULL), _blk((N, 1), _FULL),
                  _blk((1, HID), _FULL), _blk((HID, NCLS), _FULL)],
        out_specs=_blk((R, NCLS), _ROW),
        out_shape=jax.ShapeDtypeStruct((N, NCLS), f32),
    )(afinb, z1, dinvf, b1r, W2)

    x = pl.pallas_call(
        _task2_body, grid=(G,),
        in_specs=[_blk((R, N), _ROW), _blk((N, NCLS), _FULL), _blk((N, 1), _FULL),
                  _blk((1, NCLS), _FULL)],
        out_specs=_blk((R, NCLS), _ROW),
        out_shape=jax.ShapeDtypeStruct((N, NCLS), f32),
    )(afinb, v, dinvf, b2r)

    return (x, afin, anew)


# speedup vs baseline: 1.8108x; 1.8108x over previous
import jax
import jax.numpy as jnp
from jax.experimental import pallas as pl

N = 4000
F = 256
HID = 128
NCLS = 16
KSEL = 50
HALF = F // 2
R = 400
G = N // R
BISECT_ITERS = 48


def _deg_body(adj_ref, dinv_ref):
    d = jnp.sum(adj_ref[...], axis=1, keepdims=True)
    dinv_ref[...] = jnp.where(d > 0, 1.0 / jnp.sqrt(d), 0.0)


def _enc1_body(adj_ref, x_ref, t1_ref, dinv_ref, dinvT_ref, w1_ref, h_ref, z1_ref):
    i = pl.program_id(0)
    x_blk = x_ref[pl.ds(i * R, R), :]
    z1_ref[...] = jnp.dot(x_blk, w1_ref[...], preferred_element_type=jnp.float32)
    normA = adj_ref[...] * dinv_ref[pl.ds(i * R, R), :] * dinvT_ref[...]
    y = x_ref[...] * t1_ref[...]
    h_ref[...] = jnp.tanh(jnp.dot(normA, y, preferred_element_type=jnp.float32))


def _enc2_body(adj_ref, h_ref, t2_ref, dinv_ref, dinvT_ref, emb_ref):
    i = pl.program_id(0)
    normA = adj_ref[...] * dinv_ref[pl.ds(i * R, R), :] * dinvT_ref[...]
    y = h_ref[...] * t2_ref[...]
    z = jnp.dot(normA, y, preferred_element_type=jnp.float32)
    nrm = jnp.sqrt(jnp.sum(z * z, axis=1, keepdims=True)) + 1e-12
    emb_ref[...] = z / nrm


def _sim_block(e, embT_ref):
    return (jnp.dot(e[:, :HALF], embT_ref[:HALF, :], preferred_element_type=jnp.float32)
            + jnp.dot(e[:, HALF:], embT_ref[HALF:, :], preferred_element_type=jnp.float32))


def _thr_body(emb_ref, embT_ref, t_ref):
    s = _sim_block(emb_ref[...], embT_ref)
    lo = jnp.full((R, 1), -2.0, jnp.float32)
    hi = jnp.full((R, 1), 2.0, jnp.float32)
    done = jnp.zeros((R, 1), jnp.float32)

    def cond(c):
        i, _, _, done = c
        return jnp.logical_and(i < BISECT_ITERS, jnp.min(done) < 0.5)

    def body(c):
        i, lo, hi, done = c
        mid = 0.5 * (lo + hi)
        cnt = jnp.sum((s >= mid).astype(jnp.float32), axis=1, keepdims=True)
        ge = jnp.logical_and(done < 0.5, cnt >= KSEL)
        lt = jnp.logical_and(done < 0.5, cnt < KSEL)
        hit = jnp.logical_and(ge, cnt <= KSEL)
        lo2 = jnp.where(ge, mid, lo)
        hi2 = jnp.where(lt, mid, hi)
        return i + 1, lo2, hi2, jnp.maximum(done, jnp.where(hit, 1.0, 0.0))

    _, lo, hi, done = jax.lax.while_loop(cond, body, (0, lo, hi, done))
    t_ref[...] = lo


def _assemble_body(emb_ref, embT_ref, t_ref, tT_ref, conf_ref, th_ref, adj_ref,
                   anew_ref, afin_ref, afinb_ref, dinvf_ref):
    s = _sim_block(emb_ref[...], embT_ref)
    m = 0.5 * ((s >= t_ref[...]).astype(jnp.float32)
               + (s >= tT_ref[...]).astype(jnp.float32))
    a1 = s * m
    z = conf_ref[...] * (a1 > 0).astype(jnp.float32) - th_ref[...]
    w = 2.0 / (1.0 + jnp.exp(-z))
    mask = jnp.where(z >= 0, w, 0.5)
    anew = a1 * mask
    anew_ref[...] = anew
    afin = anew + adj_ref[...]
    afin_ref[...] = afin
    afinb_ref[...] = afin.astype(jnp.bfloat16)
    df = jnp.sum(afin, axis=1, keepdims=True)
    dinvf_ref[...] = jnp.where(df > 0, 1.0 / jnp.sqrt(df), 0.0)


def _task1_body(adjfb_ref, z1_ref, dinvf_ref, b1_ref, w2_ref, v_ref):
    i = pl.program_id(0)
    y = (z1_ref[...] * dinvf_ref[...]).astype(jnp.bfloat16)
    x1 = (jnp.dot(adjfb_ref[...], y, preferred_element_type=jnp.float32)
          * dinvf_ref[pl.ds(i * R, R), :] + b1_ref[...])
    x1 = jnp.maximum(x1, 0.0)
    v_ref[...] = jnp.dot(x1, w2_ref[...], preferred_element_type=jnp.float32)


def _task2_body(adjfb_ref, v_ref, dinvf_ref, b2_ref, out_ref):
    i = pl.program_id(0)
    y = (v_ref[...] * dinvf_ref[...]).astype(jnp.bfloat16)
    out_ref[...] = (jnp.dot(adjfb_ref[...], y, preferred_element_type=jnp.float32)
                    * dinvf_ref[pl.ds(i * R, R), :] + b2_ref[...])


def _blk(shape, imap):
    return pl.BlockSpec(shape, imap)


_ROW = lambda i: (i, 0)
_FULL = lambda i: (0, 0)


def kernel(input, Adj, t1, t2, thresholds, confidence_vector, W1, b1, W2, b2):
    f32 = jnp.float32
    t1r = t1.reshape(1, F)
    t2r = t2.reshape(1, F)
    b1r = b1.reshape(1, HID)
    b2r = b2.reshape(1, NCLS)
    confr = confidence_vector.reshape(1, N)

    dinv = pl.pallas_call(
        _deg_body, grid=(G,),
        in_specs=[_blk((R, N), _ROW)],
        out_specs=_blk((R, 1), _ROW),
        out_shape=jax.ShapeDtypeStruct((N, 1), f32),
    )(Adj)
    dinvT = dinv.reshape(1, N)

    h, z1 = pl.pallas_call(
        _enc1_body, grid=(G,),
        in_specs=[_blk((R, N), _ROW), _blk((N, F), _FULL), _blk((1, F), _FULL),
                  _blk((N, 1), _FULL), _blk((1, N), _FULL), _blk((F, HID), _FULL)],
        out_specs=[_blk((R, F), _ROW), _blk((R, HID), _ROW)],
        out_shape=[jax.ShapeDtypeStruct((N, F), f32),
                   jax.ShapeDtypeStruct((N, HID), f32)],
    )(Adj, input, t1r, dinv, dinvT, W1)

    emb = pl.pallas_call(
        _enc2_body, grid=(G,),
        in_specs=[_blk((R, N), _ROW), _blk((N, F), _FULL), _blk((1, F), _FULL),
                  _blk((N, 1), _FULL), _blk((1, N), _FULL)],
        out_specs=_blk((R, F), _ROW),
        out_shape=jax.ShapeDtypeStruct((N, F), f32),
    )(Adj, h, t2r, dinv, dinvT)
    embT = emb.T

    t = pl.pallas_call(
        _thr_body, grid=(G,),
        in_specs=[_blk((R, F), _ROW), _blk((F, N), _FULL)],
        out_specs=_blk((R, 1), _ROW),
        out_shape=jax.ShapeDtypeStruct((N, 1), f32),
    )(emb, embT)
    tT = t.reshape(1, N)

    anew, afin, afinb, dinvf = pl.pallas_call(
        _assemble_body, grid=(G,),
        in_specs=[_blk((R, F), _ROW), _blk((F, N), _FULL), _blk((R, 1), _ROW),
                  _blk((1, N), _FULL), _blk((1, N), _FULL), _blk((R, 1), _ROW),
                  _blk((R, N), _ROW)],
        out_specs=[_blk((R, N), _ROW), _blk((R, N), _ROW), _blk((R, N), _ROW),
                   _blk((R, 1), _ROW)],
        out_shape=[jax.ShapeDtypeStruct((N, N), f32),
                   jax.ShapeDtypeStruct((N, N), f32),
                   jax.ShapeDtypeStruct((N, N), jnp.bfloat16),
                   jax.ShapeDtypeStruct((N, 1), f32)],
    )(emb, embT, t, tT, confr, thresholds, Adj)

    v = pl.pallas_call(
        _task1_body, grid=(G,),
        in_specs=[_blk((R, N), _ROW), _blk((N, HID), _FULL), _blk((N, 1), _FULL),
                  _blk((1, HID), _FULL), _blk((HID, NCLS), _FULL)],
        out_specs=_blk((R, NCLS), _ROW),
        out_shape=jax.ShapeDtypeStruct((N, NCLS), f32),
    )(afinb, z1, dinvf, b1r, W2)

    x = pl.pallas_call(
        _task2_body, grid=(G,),
        in_specs=[_blk((R, N), _ROW), _blk((N, NCLS), _FULL), _blk((N, 1), _FULL),
                  _blk((1, NCLS), _FULL)],
        out_specs=_blk((R, NCLS), _ROW),
        out_shape=jax.ShapeDtypeStruct((N, NCLS), f32),
    )(afinb, v, dinvf, b2r)

    return (x, afin, anew)
